# Initial kernel scaffold; baseline (speedup 1.0000x reference)
import jax
import jax.numpy as jnp
from jax.experimental import pallas as pl


def kernel(x, edge_index, edge_weight, fc0_W, fc0_b, Wz, bz, Wr, br, Wh, bh, fc_W, fc_b):
    raise NotImplementedError("write your pallas kernel here")


# speedup vs baseline: 8.7604x; 8.7604x over previous
import jax
import jax.numpy as jnp
from jax import lax
from jax.experimental import pallas as pl
from jax.experimental.pallas import tpu as pltpu
from jax.experimental.pallas import tpu_sc as plsc

N = 10000
E = 320000
D_IN = 128
D_H1 = 256
D_OUT = 128

_LANES = 16
_NTILES = 16
_CH = 128
_CPT = 160
_NCHUNK = _CPT * _NTILES
_EPAD = _NCHUNK * _CH
_NPAD = N + 8
_WB = 632
_WB_LAST = N - 15 * _WB

_mesh = plsc.VectorSubcoreMesh(core_axis_name="c", subcore_axis_name="s")


def _zero_vmem_2d(ref, nrows, ncols):
    zer = jnp.zeros((_LANES,), jnp.float32)
    per_row = ncols // _LANES

    def body(i, _):
        r = i // per_row
        c = (i % per_row) * _LANES
        ref[r, pl.ds(c, _LANES)] = zer
        return 0

    lax.fori_loop(0, nrows * per_row, body, 0)


def _zero_vmem_1d(ref, n):
    zer = jnp.zeros((_LANES,), jnp.float32)

    def body(i, _):
        ref[pl.ds(i * _LANES, _LANES)] = zer
        return 0

    lax.fori_loop(0, n // _LANES, body, 0)


def _tile_spans(s):
    cbase = pl.multiple_of(_CPT * s, 8)
    rbase = pl.multiple_of(_WB * s, 8)
    return cbase, rbase


def _deg_body(w2d, row_d2d, col_d2d, dego, degi, acc, wv, iv, zb):
    c = lax.axis_index("c")
    s = lax.axis_index("s")
    cbase, rbase = _tile_spans(s)
    last = s == _NTILES - 1

    _zero_vmem_1d(zb, _WB)

    @pl.when(~last)
    def _():
        pltpu.sync_copy(zb.at[pl.ds(0, _WB)], acc.at[pl.ds(rbase, _WB)])

    @pl.when(last)
    def _():
        pltpu.sync_copy(zb.at[pl.ds(0, _WB_LAST)], acc.at[pl.ds(rbase, _WB_LAST)])

    def run(idx2d, out):
        pltpu.sync_copy(w2d.at[pl.ds(cbase, _CPT)], wv)
        pltpu.sync_copy(idx2d.at[pl.ds(cbase, _CPT)], iv)
        plsc.subcore_barrier()

        def body(i, _):
            pltpu.sync_copy(wv.at[i], acc.at[iv.at[i]], add=True)
            return 0

        lax.fori_loop(0, _CPT, body, 0)
        plsc.subcore_barrier()

        @pl.when(~last)
        def _():
            pltpu.sync_copy(acc.at[pl.ds(rbase, _WB)], zb)
            pltpu.sync_copy(zb, out.at[pl.ds(rbase, _WB)])

        @pl.when(last)
        def _():
            pltpu.sync_copy(acc.at[pl.ds(rbase, _WB_LAST)], zb.at[pl.ds(0, _WB_LAST)])
            pltpu.sync_copy(zb.at[pl.ds(0, _WB_LAST)], out.at[pl.ds(rbase, _WB_LAST)])

    @pl.when(c == 0)
    def _():
        run(row_d2d, dego)

    @pl.when(c == 1)
    def _():
        run(col_d2d, degi)


_deg_call = pl.kernel(
    _deg_body,
    out_type=[
        jax.ShapeDtypeStruct((N,), jnp.float32),
        jax.ShapeDtypeStruct((N,), jnp.float32),
    ],
    mesh=_mesh,
    scratch_types=[
        pltpu.VMEM_SHARED((_NPAD,), jnp.float32),
        pltpu.VMEM((_CPT, _CH), jnp.float32),
        pltpu.VMEM((_CPT, _CH), jnp.int32),
        pltpu.VMEM((_WB,), jnp.float32),
    ],
)


_GRP = 32


def _seg_body(t1, t2, src2d, dst2d, o1, o2, acc, sidx, didx, rows, gsem):
    c = lax.axis_index("c")
    s = lax.axis_index("s")
    cbase, rbase = _tile_spans(s)
    last = s == _NTILES - 1

    _zero_vmem_2d(rows, _CH, D_OUT)

    def zero_rows(total):
        for j in range(total // _CH):
            off = pl.multiple_of(rbase + j * _CH, 8)
            pltpu.sync_copy(rows, acc.at[pl.ds(off, _CH)])
        rem = total % _CH
        if rem:
            off = pl.multiple_of(rbase + (total // _CH) * _CH, 8)
            pltpu.sync_copy(rows.at[pl.ds(0, rem)], acc.at[pl.ds(off, rem)])

    @pl.when(~last)
    def _():
        zero_rows(_WB)

    @pl.when(last)
    def _():
        zero_rows(_WB_LAST)

    def run(tab, out):
        plsc.subcore_barrier()

        def outer(g, _):
            off = pl.multiple_of(cbase + g * _GRP, 8)
            pltpu.sync_copy(src2d.at[pl.ds(off, _GRP)], sidx)
            pltpu.sync_copy(dst2d.at[pl.ds(off, _GRP)], didx)

            def body(i, _):
                pltpu.async_copy(tab.at[sidx.at[i]], rows, gsem).wait()
                pltpu.sync_copy(rows, acc.at[didx.at[i]], add=True)
                return 0

            lax.fori_loop(0, _GRP, body, 0)
            return 0

        lax.fori_loop(0, _CPT // _GRP, outer, 0)
        plsc.subcore_barrier()

        @pl.when(~last)
        def _():
            pltpu.sync_copy(acc.at[pl.ds(rbase, _WB)], out.at[pl.ds(rbase, _WB)])

        @pl.when(last)
        def _():
            pltpu.sync_copy(acc.at[pl.ds(rbase, _WB_LAST)],
                            out.at[pl.ds(rbase, _WB_LAST)])

    @pl.when(c == 0)
    def _():
        run(t1, o1)

    @pl.when(c == 1)
    def _():
        run(t2, o2)


_seg_call = pl.kernel(
    _seg_body,
    out_type=[
        jax.ShapeDtypeStruct((N, D_OUT), jnp.float32),
        jax.ShapeDtypeStruct((N, D_OUT), jnp.float32),
    ],
    mesh=_mesh,
    scratch_types=[
        pltpu.VMEM_SHARED((_NPAD, D_OUT), jnp.float32),
        pltpu.VMEM((_GRP, _CH), jnp.int32),
        pltpu.VMEM((_GRP, _CH), jnp.int32),
        pltpu.VMEM((_CH, D_OUT), jnp.float32),
        pltpu.SemaphoreType.DMA,
    ],
)


_RB = 1000


def _pre_body(x, dego, f0w, f0b, wo, wi, bc, p1, p2, q1, q2, base):
    xf = jnp.maximum(
        jnp.dot(x[...], f0w[...], preferred_element_type=jnp.float32, precision=lax.Precision.HIGHEST) + f0b[...], 0.0)
    xs = xf * (1.0 / dego[...])
    p = jnp.dot(xs, wo[...], preferred_element_type=jnp.float32, precision=lax.Precision.HIGHEST)
    q = jnp.dot(xf, wi[...], preferred_element_type=jnp.float32, precision=lax.Precision.HIGHEST)
    p1[...] = p[:, :D_OUT]
    p2[...] = p[:, D_OUT:]
    q1[...] = q[:, :D_OUT]
    q2[...] = q[:, D_OUT:]
    base[...] = jnp.dot(xf, bc[...], preferred_element_type=jnp.float32, precision=lax.Precision.HIGHEST)


_pre_call = pl.pallas_call(
    _pre_body,
    grid=(N // _RB,),
    in_specs=[
        pl.BlockSpec((_RB, D_IN), lambda i: (i, 0)),
        pl.BlockSpec((_RB, 1), lambda i: (i, 0)),
        pl.BlockSpec((D_IN, D_H1), lambda i: (0, 0)),
        pl.BlockSpec((1, D_H1), lambda i: (0, 0)),
        pl.BlockSpec((D_H1, D_H1), lambda i: (0, 0)),
        pl.BlockSpec((D_H1, D_H1), lambda i: (0, 0)),
        pl.BlockSpec((D_H1, D_H1), lambda i: (0, 0)),
    ],
    out_specs=[
        pl.BlockSpec((_RB, D_OUT), lambda i: (i, 0)),
        pl.BlockSpec((_RB, D_OUT), lambda i: (i, 0)),
        pl.BlockSpec((_RB, D_OUT), lambda i: (i, 0)),
        pl.BlockSpec((_RB, D_OUT), lambda i: (i, 0)),
        pl.BlockSpec((_RB, D_H1), lambda i: (i, 0)),
    ],
    out_shape=[
        jax.ShapeDtypeStruct((N, D_OUT), jnp.float32),
        jax.ShapeDtypeStruct((N, D_OUT), jnp.float32),
        jax.ShapeDtypeStruct((N, D_OUT), jnp.float32),
        jax.ShapeDtypeStruct((N, D_OUT), jnp.float32),
        jax.ShapeDtypeStruct((N, D_H1), jnp.float32),
    ],
)


def _post_body(base, ao1, ao2, ai1, ai2, degi, bz, bh, fcw, h_ref, o_ref):
    invdi = 1.0 / degi[...]
    zpre = base[:, :D_OUT] + ao1[...] + invdi * ai1[...] + bz[...]
    hpre = base[:, D_OUT:] + ao2[...] + invdi * ai2[...] + bh[...]
    z = jax.nn.sigmoid(zpre)
    ht = jnp.tanh(hpre)
    h = (1.0 - z) * ht
    h_ref[...] = h
    o_ref[...] = jnp.sum(jnp.maximum(h, 0.0) * fcw[...], axis=1, keepdims=True)


_post_call = pl.pallas_call(
    _post_body,
    grid=(N // _RB,),
    in_specs=[
        pl.BlockSpec((_RB, D_H1), lambda i: (i, 0)),
        pl.BlockSpec((_RB, D_OUT), lambda i: (i, 0)),
        pl.BlockSpec((_RB, D_OUT), lambda i: (i, 0)),
        pl.BlockSpec((_RB, D_OUT), lambda i: (i, 0)),
        pl.BlockSpec((_RB, D_OUT), lambda i: (i, 0)),
        pl.BlockSpec((_RB, 1), lambda i: (i, 0)),
        pl.BlockSpec((1, D_OUT), lambda i: (0, 0)),
        pl.BlockSpec((1, D_OUT), lambda i: (0, 0)),
        pl.BlockSpec((1, D_OUT), lambda i: (0, 0)),
    ],
    out_specs=[
        pl.BlockSpec((_RB, D_OUT), lambda i: (i, 0)),
        pl.BlockSpec((_RB, 1), lambda i: (i, 0)),
    ],
    out_shape=[
        jax.ShapeDtypeStruct((N, D_OUT), jnp.float32),
        jax.ShapeDtypeStruct((N, 1), jnp.float32),
    ],
)


def kernel(x, edge_index, edge_weight, fc0_W, fc0_b, Wz, bz, Wr, br, Wh, bh, fc_W, fc_b):
    pad0 = jnp.zeros((_EPAD - E,), jnp.int32)
    padn = jnp.full((_EPAD - E,), N, jnp.int32)
    row_s = jnp.concatenate([edge_index[0], pad0]).reshape(_NCHUNK, _CH)
    row_d = jnp.concatenate([edge_index[0], padn]).reshape(_NCHUNK, _CH)
    col_s = jnp.concatenate([edge_index[1], pad0]).reshape(_NCHUNK, _CH)
    col_d = jnp.concatenate([edge_index[1], padn]).reshape(_NCHUNK, _CH)
    w2d = jnp.concatenate(
        [edge_weight, jnp.zeros((_EPAD - E,), jnp.float32)]).reshape(_NCHUNK, _CH)

    wo = jnp.concatenate([Wz[0, 1][:D_H1], Wh[0, 1][:D_H1]], axis=1)
    wi = jnp.concatenate([Wz[1, 1][:D_H1], Wh[1, 1][:D_H1]], axis=1)
    bc = jnp.concatenate([Wz[0, 0][:D_H1] + Wz[1, 0][:D_H1],
                          Wh[0, 0][:D_H1] + Wh[1, 0][:D_H1]], axis=1)

    deg_out, deg_in = _deg_call(w2d, row_d, col_d)

    p1, p2, q1, q2, base = _pre_call(
        x, deg_out.reshape(N, 1), fc0_W, fc0_b.reshape(1, D_H1), wo, wi, bc)

    ao1, ao2 = _seg_call(p1, p2, row_s, col_d)
    ai1, ai2 = _seg_call(q1, q2, col_s, row_d)

    h, o = _post_call(base, ao1, ao2, ai1, ai2, deg_in.reshape(N, 1),
                      bz.reshape(1, D_OUT), bh.reshape(1, D_OUT),
                      fc_W.reshape(1, D_OUT))
    out = o[:, 0] + fc_b[0]
    return out, h


# speedup vs baseline: 10.5171x; 1.2005x over previous
import jax
import jax.numpy as jnp
from jax import lax
from jax.experimental import pallas as pl
from jax.experimental.pallas import tpu as pltpu
from jax.experimental.pallas import tpu_sc as plsc

N = 10000
E = 320000
D_IN = 128
D_H1 = 256
D_OUT = 128

_LANES = 16
_NTILES = 16
_CH = 128
_CPT = 160
_NCHUNK = _CPT * _NTILES
_EPAD = _NCHUNK * _CH
_NPAD = N + 8
_WB = 632
_WB_LAST = N - 15 * _WB

_mesh = plsc.VectorSubcoreMesh(core_axis_name="c", subcore_axis_name="s")


def _zero_vmem_2d(ref, nrows, ncols):
    zer = jnp.zeros((_LANES,), jnp.float32)
    per_row = ncols // _LANES

    def body(i, _):
        r = i // per_row
        c = (i % per_row) * _LANES
        ref[r, pl.ds(c, _LANES)] = zer
        return 0

    lax.fori_loop(0, nrows * per_row, body, 0)


def _zero_vmem_1d(ref, n):
    zer = jnp.zeros((_LANES,), jnp.float32)

    def body(i, _):
        ref[pl.ds(i * _LANES, _LANES)] = zer
        return 0

    lax.fori_loop(0, n // _LANES, body, 0)


def _tile_spans(s):
    cbase = pl.multiple_of(_CPT * s, 8)
    rbase = pl.multiple_of(_WB * s, 8)
    return cbase, rbase


def _deg_body(w2d, row_d2d, col_d2d, dego, degi, acc, wv, iv, zb):
    c = lax.axis_index("c")
    s = lax.axis_index("s")
    cbase, rbase = _tile_spans(s)
    last = s == _NTILES - 1

    _zero_vmem_1d(zb, _WB)

    @pl.when(~last)
    def _():
        pltpu.sync_copy(zb.at[pl.ds(0, _WB)], acc.at[pl.ds(rbase, _WB)])

    @pl.when(last)
    def _():
        pltpu.sync_copy(zb.at[pl.ds(0, _WB_LAST)], acc.at[pl.ds(rbase, _WB_LAST)])

    def run(idx2d, out):
        pltpu.sync_copy(w2d.at[pl.ds(cbase, _CPT)], wv)
        pltpu.sync_copy(idx2d.at[pl.ds(cbase, _CPT)], iv)
        plsc.subcore_barrier()

        def body(i, _):
            pltpu.sync_copy(wv.at[i], acc.at[iv.at[i]], add=True)
            return 0

        lax.fori_loop(0, _CPT, body, 0)
        plsc.subcore_barrier()

        @pl.when(~last)
        def _():
            pltpu.sync_copy(acc.at[pl.ds(rbase, _WB)], zb)
            pltpu.sync_copy(zb, out.at[pl.ds(rbase, _WB)])

        @pl.when(last)
        def _():
            pltpu.sync_copy(acc.at[pl.ds(rbase, _WB_LAST)], zb.at[pl.ds(0, _WB_LAST)])
            pltpu.sync_copy(zb.at[pl.ds(0, _WB_LAST)], out.at[pl.ds(rbase, _WB_LAST)])

    @pl.when(c == 0)
    def _():
        run(row_d2d, dego)

    @pl.when(c == 1)
    def _():
        run(col_d2d, degi)


_deg_call = pl.kernel(
    _deg_body,
    out_type=[
        jax.ShapeDtypeStruct((N,), jnp.float32),
        jax.ShapeDtypeStruct((N,), jnp.float32),
    ],
    mesh=_mesh,
    scratch_types=[
        pltpu.VMEM_SHARED((_NPAD,), jnp.float32),
        pltpu.VMEM((_CPT, _CH), jnp.float32),
        pltpu.VMEM((_CPT, _CH), jnp.int32),
        pltpu.VMEM((_WB,), jnp.float32),
    ],
)


_GRP = 32


def _seg_body(t1, t2, src2d, dst2d, o1, o2, acc, sidx, didx, rows0, rows1,
              gsem0, gsem1):
    c = lax.axis_index("c")
    s = lax.axis_index("s")
    cbase, rbase = _tile_spans(s)
    last = s == _NTILES - 1

    _zero_vmem_2d(rows0, _CH, D_OUT)

    def zero_rows(total):
        for j in range(total // _CH):
            off = pl.multiple_of(rbase + j * _CH, 8)
            pltpu.sync_copy(rows0, acc.at[pl.ds(off, _CH)])
        rem = total % _CH
        if rem:
            off = pl.multiple_of(rbase + (total // _CH) * _CH, 8)
            pltpu.sync_copy(rows0.at[pl.ds(0, rem)], acc.at[pl.ds(off, rem)])

    @pl.when(~last)
    def _():
        zero_rows(_WB)

    @pl.when(last)
    def _():
        zero_rows(_WB_LAST)

    def run(tab, out):
        plsc.subcore_barrier()

        def outer(g, _):
            off = pl.multiple_of(cbase + g * _GRP, 8)
            pltpu.sync_copy(src2d.at[pl.ds(off, _GRP)], sidx)
            pltpu.sync_copy(dst2d.at[pl.ds(off, _GRP)], didx)
            pltpu.async_copy(tab.at[sidx.at[0]], rows0, gsem0)

            def pair(k, _):
                i0 = 2 * k
                pltpu.async_copy(tab.at[sidx.at[i0 + 1]], rows1, gsem1)
                pltpu.make_async_copy(tab.at[sidx.at[i0]], rows0, gsem0).wait()
                pltpu.sync_copy(rows0, acc.at[didx.at[i0]], add=True)

                @pl.when(k < _GRP // 2 - 1)
                def _():
                    pltpu.async_copy(tab.at[sidx.at[i0 + 2]], rows0, gsem0)

                pltpu.make_async_copy(tab.at[sidx.at[i0 + 1]], rows1, gsem1).wait()
                pltpu.sync_copy(rows1, acc.at[didx.at[i0 + 1]], add=True)
                return 0

            lax.fori_loop(0, _GRP // 2, pair, 0)
            return 0

        lax.fori_loop(0, _CPT // _GRP, outer, 0)
        plsc.subcore_barrier()

        @pl.when(~last)
        def _():
            pltpu.sync_copy(acc.at[pl.ds(rbase, _WB)], out.at[pl.ds(rbase, _WB)])

        @pl.when(last)
        def _():
            pltpu.sync_copy(acc.at[pl.ds(rbase, _WB_LAST)],
                            out.at[pl.ds(rbase, _WB_LAST)])

    @pl.when(c == 0)
    def _():
        run(t1, o1)

    @pl.when(c == 1)
    def _():
        run(t2, o2)


_seg_call = pl.kernel(
    _seg_body,
    out_type=[
        jax.ShapeDtypeStruct((N, D_OUT), jnp.float32),
        jax.ShapeDtypeStruct((N, D_OUT), jnp.float32),
    ],
    mesh=_mesh,
    scratch_types=[
        pltpu.VMEM_SHARED((_NPAD, D_OUT), jnp.float32),
        pltpu.VMEM((_GRP, _CH), jnp.int32),
        pltpu.VMEM((_GRP, _CH), jnp.int32),
        pltpu.VMEM((_CH, D_OUT), jnp.float32),
        pltpu.VMEM((_CH, D_OUT), jnp.float32),
        pltpu.SemaphoreType.DMA,
        pltpu.SemaphoreType.DMA,
    ],
)


_RB = 1000


def _pre_body(x, dego, f0w, f0b, wo, wi, bc, p1, p2, q1, q2, base):
    xf = jnp.maximum(
        jnp.dot(x[...], f0w[...], preferred_element_type=jnp.float32, precision=lax.Precision.HIGHEST) + f0b[...], 0.0)
    xs = xf * (1.0 / dego[...])
    p = jnp.dot(xs, wo[...], preferred_element_type=jnp.float32, precision=lax.Precision.HIGHEST)
    q = jnp.dot(xf, wi[...], preferred_element_type=jnp.float32, precision=lax.Precision.HIGHEST)
    p1[...] = p[:, :D_OUT]
    p2[...] = p[:, D_OUT:]
    q1[...] = q[:, :D_OUT]
    q2[...] = q[:, D_OUT:]
    base[...] = jnp.dot(xf, bc[...], preferred_element_type=jnp.float32, precision=lax.Precision.HIGHEST)


_pre_call = pl.pallas_call(
    _pre_body,
    grid=(N // _RB,),
    in_specs=[
        pl.BlockSpec((_RB, D_IN), lambda i: (i, 0)),
        pl.BlockSpec((_RB, 1), lambda i: (i, 0)),
        pl.BlockSpec((D_IN, D_H1), lambda i: (0, 0)),
        pl.BlockSpec((1, D_H1), lambda i: (0, 0)),
        pl.BlockSpec((D_H1, D_H1), lambda i: (0, 0)),
        pl.BlockSpec((D_H1, D_H1), lambda i: (0, 0)),
        pl.BlockSpec((D_H1, D_H1), lambda i: (0, 0)),
    ],
    out_specs=[
        pl.BlockSpec((_RB, D_OUT), lambda i: (i, 0)),
        pl.BlockSpec((_RB, D_OUT), lambda i: (i, 0)),
        pl.BlockSpec((_RB, D_OUT), lambda i: (i, 0)),
        pl.BlockSpec((_RB, D_OUT), lambda i: (i, 0)),
        pl.BlockSpec((_RB, D_H1), lambda i: (i, 0)),
    ],
    out_shape=[
        jax.ShapeDtypeStruct((N, D_OUT), jnp.float32),
        jax.ShapeDtypeStruct((N, D_OUT), jnp.float32),
        jax.ShapeDtypeStruct((N, D_OUT), jnp.float32),
        jax.ShapeDtypeStruct((N, D_OUT), jnp.float32),
        jax.ShapeDtypeStruct((N, D_H1), jnp.float32),
    ],
)


def _post_body(base, ao1, ao2, ai1, ai2, degi, bz, bh, fcw, h_ref, o_ref):
    invdi = 1.0 / degi[...]
    zpre = base[:, :D_OUT] + ao1[...] + invdi * ai1[...] + bz[...]
    hpre = base[:, D_OUT:] + ao2[...] + invdi * ai2[...] + bh[...]
    z = jax.nn.sigmoid(zpre)
    ht = jnp.tanh(hpre)
    h = (1.0 - z) * ht
    h_ref[...] = h
    o_ref[...] = jnp.sum(jnp.maximum(h, 0.0) * fcw[...], axis=1, keepdims=True)


_post_call = pl.pallas_call(
    _post_body,
    grid=(N // _RB,),
    in_specs=[
        pl.BlockSpec((_RB, D_H1), lambda i: (i, 0)),
        pl.BlockSpec((_RB, D_OUT), lambda i: (i, 0)),
        pl.BlockSpec((_RB, D_OUT), lambda i: (i, 0)),
        pl.BlockSpec((_RB, D_OUT), lambda i: (i, 0)),
        pl.BlockSpec((_RB, D_OUT), lambda i: (i, 0)),
        pl.BlockSpec((_RB, 1), lambda i: (i, 0)),
        pl.BlockSpec((1, D_OUT), lambda i: (0, 0)),
        pl.BlockSpec((1, D_OUT), lambda i: (0, 0)),
        pl.BlockSpec((1, D_OUT), lambda i: (0, 0)),
    ],
    out_specs=[
        pl.BlockSpec((_RB, D_OUT), lambda i: (i, 0)),
        pl.BlockSpec((_RB, 1), lambda i: (i, 0)),
    ],
    out_shape=[
        jax.ShapeDtypeStruct((N, D_OUT), jnp.float32),
        jax.ShapeDtypeStruct((N, 1), jnp.float32),
    ],
)


def kernel(x, edge_index, edge_weight, fc0_W, fc0_b, Wz, bz, Wr, br, Wh, bh, fc_W, fc_b):
    pad0 = jnp.zeros((_EPAD - E,), jnp.int32)
    padn = jnp.full((_EPAD - E,), N, jnp.int32)
    row_s = jnp.concatenate([edge_index[0], pad0]).reshape(_NCHUNK, _CH)
    row_d = jnp.concatenate([edge_index[0], padn]).reshape(_NCHUNK, _CH)
    col_s = jnp.concatenate([edge_index[1], pad0]).reshape(_NCHUNK, _CH)
    col_d = jnp.concatenate([edge_index[1], padn]).reshape(_NCHUNK, _CH)
    w2d = jnp.concatenate(
        [edge_weight, jnp.zeros((_EPAD - E,), jnp.float32)]).reshape(_NCHUNK, _CH)

    wo = jnp.concatenate([Wz[0, 1][:D_H1], Wh[0, 1][:D_H1]], axis=1)
    wi = jnp.concatenate([Wz[1, 1][:D_H1], Wh[1, 1][:D_H1]], axis=1)
    bc = jnp.concatenate([Wz[0, 0][:D_H1] + Wz[1, 0][:D_H1],
                          Wh[0, 0][:D_H1] + Wh[1, 0][:D_H1]], axis=1)

    deg_out, deg_in = _deg_call(w2d, row_d, col_d)

    p1, p2, q1, q2, base = _pre_call(
        x, deg_out.reshape(N, 1), fc0_W, fc0_b.reshape(1, D_H1), wo, wi, bc)

    ao1, ao2 = _seg_call(p1, p2, row_s, col_d)
    ai1, ai2 = _seg_call(q1, q2, col_s, row_d)

    h, o = _post_call(base, ao1, ao2, ai1, ai2, deg_in.reshape(N, 1),
                      bz.reshape(1, D_OUT), bh.reshape(1, D_OUT),
                      fc_W.reshape(1, D_OUT))
    out = o[:, 0] + fc_b[0]
    return out, h
